# Initial kernel scaffold; baseline (speedup 1.0000x reference)
import jax
import jax.numpy as jnp
from jax.experimental import pallas as pl


def kernel(x, edge_index, W1, b1, W2, b2, W3, b3):
    raise NotImplementedError("write your pallas kernel here")


# speedup vs baseline: 15.1997x; 15.1997x over previous
import functools

import jax
import jax.numpy as jnp
from jax import lax
from jax.experimental import pallas as pl
from jax.experimental.pallas import tpu as pltpu
from jax.experimental.pallas import tpu_sc as plsc

N = 10000
NP = 10240
D = 128
E = 320000

CHUNK = 80
NC, NS = 2, 16
NW = NC * NS
NCHUNK = E // (NW * CHUNK)
ROWS_PT = NP // NS

_MESH = plsc.VectorSubcoreMesh(core_axis_name="c", subcore_axis_name="s")


@functools.partial(
    pl.kernel,
    out_type=jax.ShapeDtypeStruct((NC, NP, D), jnp.float32),
    mesh=_MESH,
    scratch_types=[
        pltpu.VMEM((NCHUNK, CHUNK), jnp.int32),
        pltpu.VMEM((NCHUNK, CHUNK), jnp.int32),
        pltpu.VMEM((CHUNK, D), jnp.float32),
        pltpu.VMEM_SHARED((NP, D), jnp.float32),
    ],
)
def _edge_pass(g_hbm, row_hbm, col_hbm, p_hbm, row_v, col_v, buf, acc):
    cid = lax.axis_index("c")
    sid = lax.axis_index("s")
    wid = cid * NS + sid
    pltpu.sync_copy(row_hbm.at[wid], row_v)
    pltpu.sync_copy(col_hbm.at[wid], col_v)
    sl = pl.ds(sid * ROWS_PT, ROWS_PT)
    pltpu.sync_copy(g_hbm.at[sl], acc.at[sl])
    plsc.subcore_barrier()

    @pl.loop(0, NCHUNK)
    def _(j):
        pltpu.sync_copy(g_hbm.at[row_v.at[j]], buf)
        pltpu.sync_copy(buf, acc.at[col_v.at[j]], add=True)

    plsc.subcore_barrier()
    pltpu.sync_copy(acc.at[sl], p_hbm.at[cid].at[sl])


@functools.partial(
    pl.kernel,
    out_type=jax.ShapeDtypeStruct((NC, NP, D), jnp.float32),
    mesh=_MESH,
    scratch_types=[
        pltpu.VMEM((NCHUNK, CHUNK), jnp.int32),
        pltpu.VMEM((CHUNK, D), jnp.float32),
        pltpu.VMEM_SHARED((NP, D), jnp.float32),
    ],
)
def _deg_pass(ones_hbm, col_hbm, d_hbm, col_v, ones_v, acc):
    cid = lax.axis_index("c")
    sid = lax.axis_index("s")
    wid = cid * NS + sid
    pltpu.sync_copy(col_hbm.at[wid], col_v)
    pltpu.sync_copy(ones_hbm.at[pl.ds(0, CHUNK)], ones_v)
    sl = pl.ds(sid * ROWS_PT, ROWS_PT)
    pltpu.sync_copy(ones_hbm.at[sl], acc.at[sl])
    plsc.subcore_barrier()

    @pl.loop(0, NCHUNK)
    def _(j):
        pltpu.sync_copy(ones_v, acc.at[col_v.at[j]], add=True)

    plsc.subcore_barrier()
    pltpu.sync_copy(acc.at[sl], d_hbm.at[cid].at[sl])


_BR = 640


def _dinv_of(d_ref):
    deg = d_ref[0, :, 0:1] + d_ref[1, :, 0:1] - 1.0
    return lax.rsqrt(deg)


def _tc_first_body(x_ref, wt_ref, d_ref, g_ref):
    dinv = _dinv_of(d_ref)
    h = jnp.dot(x_ref[...], wt_ref[...], preferred_element_type=jnp.float32)
    g_ref[...] = h * dinv


def _tc_mid_body(p_ref, g_ref, d_ref, b_ref, wt_ref, go_ref):
    dinv = _dinv_of(d_ref)
    u = dinv * (p_ref[0] + p_ref[1] - g_ref[...]) + b_ref[...]
    y = jnp.maximum(u, 0.0)
    go_ref[...] = jnp.dot(y, wt_ref[...], preferred_element_type=jnp.float32) * dinv


def _tc_final_body(p_ref, g_ref, d_ref, b_ref, o_ref):
    dinv = _dinv_of(d_ref)
    o_ref[...] = dinv * (p_ref[0] + p_ref[1] - g_ref[...]) + b_ref[...]


_rows_spec = pl.BlockSpec((_BR, D), lambda i: (i, 0))
_p_spec = pl.BlockSpec((NC, _BR, D), lambda i: (0, i, 0))
_deg_spec = pl.BlockSpec((NC, _BR, D), lambda i: (0, i, 0))
_w_spec = pl.BlockSpec((D, D), lambda i: (0, 0))
_b_spec = pl.BlockSpec((1, D), lambda i: (0, 0))
_out_f32 = jax.ShapeDtypeStruct((NP, D), jnp.float32)

_tc_first = pl.pallas_call(
    _tc_first_body, grid=(NP // _BR,),
    in_specs=[_rows_spec, _w_spec, _deg_spec],
    out_specs=_rows_spec, out_shape=_out_f32)

_tc_mid = pl.pallas_call(
    _tc_mid_body, grid=(NP // _BR,),
    in_specs=[_p_spec, _rows_spec, _deg_spec, _b_spec, _w_spec],
    out_specs=_rows_spec, out_shape=_out_f32)

_tc_final = pl.pallas_call(
    _tc_final_body, grid=(NP // _BR,),
    in_specs=[_p_spec, _rows_spec, _deg_spec, _b_spec],
    out_specs=_rows_spec, out_shape=_out_f32)


def kernel(x, edge_index, W1, b1, W2, b2, W3, b3):
    row3d = edge_index[0].reshape(NW, NCHUNK, CHUNK)
    col3d = edge_index[1].reshape(NW, NCHUNK, CHUNK)
    ones = jnp.ones((NP, D), jnp.float32)
    xp = jnp.zeros((NP, D), x.dtype).at[:N].set(x)

    d = _deg_pass(ones, col3d)
    g1 = _tc_first(xp, W1.T, d)
    p = _edge_pass(g1, row3d, col3d)
    g2 = _tc_mid(p, g1, d, b1.reshape(1, D), W2.T)
    p = _edge_pass(g2, row3d, col3d)
    g3 = _tc_mid(p, g2, d, b2.reshape(1, D), W3.T)
    p = _edge_pass(g3, row3d, col3d)
    return _tc_final(p, g3, d, b3.reshape(1, D))[:N]
